# baseline (speedup 1.0000x reference)
import jax
import jax.numpy as jnp
from jax import lax
from jax.experimental import pallas as pl
from jax.experimental.pallas import tpu as pltpu
from jax.experimental.pallas import tpu_sc as plsc

N = 10000
E = 320000
D = 128
HID = 128
OUT = 10
G = 64

NC = 2
NS = 16
NW = NC * NS

CHUNK = 128
TOT_CHUNKS = E // CHUNK
CH_LO = TOT_CHUNKS // NW
EXTRA = TOT_CHUNKS - NW * CH_LO
NBUF = 3
AGG_ROWS = 10112
ROWS_PER_TILE = AGG_ROWS // NS


def _sc_agg_body(x_hbm, idx_hbm, zeros_hbm, out_hbm,
                 idx_v, rows_v, agg_sh, gsem, ssem, isem):
    c = lax.axis_index("c")
    s = lax.axis_index("s")
    wid = c * NS + s
    r0 = s * ROWS_PER_TILE
    nch = CH_LO + jnp.where(wid < EXTRA, 1, 0)
    base = wid * CH_LO + jnp.minimum(wid, EXTRA)

    def idx_desc(i):
        return pltpu.make_async_copy(idx_hbm.at[base + i],
                                     idx_v.at[lax.rem(i, NBUF)],
                                     isem.at[lax.rem(i, NBUF)])

    def g_desc(i):
        rb = lax.rem(i, NBUF)
        return pltpu.make_async_copy(x_hbm.at[idx_v.at[rb, 0]],
                                     rows_v.at[rb], gsem.at[rb])

    def s_start(i):
        rb = lax.rem(i, NBUF)
        pltpu.async_copy(rows_v.at[rb], agg_sh.at[idx_v.at[rb, 1]],
                         ssem.at[rb], add=True)

    def s_wait(i):
        rb = lax.rem(i, NBUF)
        pltpu.make_async_copy(rows_v.at[rb], agg_sh.at[idx_v.at[rb, 1]],
                              ssem.at[rb]).wait()

    for b in range(NBUF):
        idx_desc(b).start()

    pltpu.sync_copy(zeros_hbm, rows_v.at[0])
    for k in range(4):
        pltpu.sync_copy(rows_v.at[0],
                        agg_sh.at[pl.ds(r0 + k * CHUNK, CHUNK)])
    pltpu.sync_copy(rows_v.at[0, pl.ds(0, ROWS_PER_TILE - 4 * CHUNK)],
                    agg_sh.at[pl.ds(r0 + 4 * CHUNK,
                                    ROWS_PER_TILE - 4 * CHUNK)])

    idx_desc(0).wait()
    g_desc(0).start()
    plsc.subcore_barrier()

    def step(i, carry):
        g_desc(i).wait()
        s_start(i)

        @pl.when(i > 0)
        def _():
            s_wait(i - 1)

        @pl.when(jnp.logical_and(i >= 1, i + 2 < nch))
        def _():
            idx_desc(i + 2).start()

        @pl.when(i + 1 < nch)
        def _():
            idx_desc(i + 1).wait()
            g_desc(i + 1).start()

        return carry

    lax.fori_loop(0, nch, step, 0)
    s_wait(nch - 1)
    plsc.subcore_barrier()

    pltpu.sync_copy(agg_sh.at[pl.ds(r0, ROWS_PER_TILE)],
                    out_hbm.at[c].at[pl.ds(r0, ROWS_PER_TILE)])


_sc_agg = pl.kernel(
    _sc_agg_body,
    out_type=jax.ShapeDtypeStruct((NC, AGG_ROWS, D), jnp.float32),
    mesh=plsc.VectorSubcoreMesh(core_axis_name="c", subcore_axis_name="s",
                                num_cores=NC, num_subcores=NS),
    scratch_types=[
        pltpu.VMEM((NBUF, 2, CHUNK), jnp.int32),
        pltpu.VMEM((NBUF, CHUNK, D), jnp.float32),
        pltpu.VMEM_SHARED((AGG_ROWS, D), jnp.float32),
        pltpu.SemaphoreType.DMA((NBUF,)),
        pltpu.SemaphoreType.DMA((NBUF,)),
        pltpu.SemaphoreType.DMA((NBUF,)),
    ],
)


def _dense1_body(aggp_ref, x_ref, wrel_ref, b_ref, wroot_ref, g_ref, be_ref,
                 out_ref):
    agg = aggp_ref[0, :N, :] + aggp_ref[1, :N, :]
    x = x_ref[...]
    y = lax.dot_general(agg, wrel_ref[...], (((1,), (1,)), ((), ())),
                        preferred_element_type=jnp.float32)
    y += lax.dot_general(x, wroot_ref[...], (((1,), (1,)), ((), ())),
                         preferred_element_type=jnp.float32)
    y += b_ref[...][None, :]
    mean = jnp.mean(y, axis=0, keepdims=True)
    var = jnp.mean((y - mean) * (y - mean), axis=0, keepdims=True)
    yn = (y - mean) * lax.rsqrt(var + 1e-5)
    yn = yn * g_ref[...][None, :] + be_ref[...][None, :]
    out_ref[...] = jnp.maximum(yn, 0.0)


_dense1 = pl.pallas_call(
    _dense1_body,
    out_shape=jax.ShapeDtypeStruct((N, HID), jnp.float32),
)


def _dense2_body(aggp_ref, h_ref, batch_ref, wrel_ref, b_ref, wroot_ref,
                 g_ref, be_ref, wc_ref, bc_ref, logits_ref, h2_ref):
    agg = aggp_ref[0, :N, :] + aggp_ref[1, :N, :]
    h = h_ref[...]
    y = lax.dot_general(agg, wrel_ref[...], (((1,), (1,)), ((), ())),
                        preferred_element_type=jnp.float32)
    y += lax.dot_general(h, wroot_ref[...], (((1,), (1,)), ((), ())),
                         preferred_element_type=jnp.float32)
    y += b_ref[...][None, :]
    mean = jnp.mean(y, axis=0, keepdims=True)
    var = jnp.mean((y - mean) * (y - mean), axis=0, keepdims=True)
    yn = (y - mean) * lax.rsqrt(var + 1e-5)
    yn = yn * g_ref[...][None, :] + be_ref[...][None, :]
    h2 = jnp.maximum(yn, 0.0)
    h2_ref[...] = h2

    gids = lax.broadcasted_iota(jnp.int32, (G, N), 0)
    onehot = jnp.where(batch_ref[...] == gids, 1.0, 0.0)
    pooled = lax.dot_general(onehot, h2, (((1,), (0,)), ((), ())),
                             preferred_element_type=jnp.float32)
    logits = lax.dot_general(pooled, wc_ref[...], (((1,), (1,)), ((), ())),
                             preferred_element_type=jnp.float32)
    logits_ref[...] = logits + bc_ref[...][None, :]


_dense2 = pl.pallas_call(
    _dense2_body,
    out_shape=[
        jax.ShapeDtypeStruct((G, OUT), jnp.float32),
        jax.ShapeDtypeStruct((N, HID), jnp.float32),
    ],
)


@jax.jit
def kernel(x, edge_index, batch, W1_rel, b1, W1_root, g1, be1,
           W2_rel, b2, W2_root, g2, be2, Wc, bc):
    idx_all = jnp.stack(
        [edge_index[0].reshape(TOT_CHUNKS, CHUNK),
         edge_index[1].reshape(TOT_CHUNKS, CHUNK)], axis=1)
    zeros = jnp.zeros((CHUNK, D), jnp.float32)

    aggp1 = _sc_agg(x, idx_all, zeros)
    h1 = _dense1(aggp1, x, W1_rel, b1, W1_root, g1, be1)
    aggp2 = _sc_agg(h1, idx_all, zeros)
    logits, h2 = _dense2(aggp2, h1, batch.reshape(1, N), W2_rel, b2,
                         W2_root, g2, be2, Wc, bc)
    return (logits, h2)


# speedup vs baseline: 1.1662x; 1.1662x over previous
import jax
import jax.numpy as jnp
from jax import lax
from jax.experimental import pallas as pl
from jax.experimental.pallas import tpu as pltpu
from jax.experimental.pallas import tpu_sc as plsc

N = 10000
E = 320000
D = 128
HID = 128
OUT = 10
G = 64

NC = 2
NS = 16
NW = NC * NS

CHUNK = 128
TOT_CHUNKS = E // CHUNK
BLK = 4
TOT_BLKS = TOT_CHUNKS // BLK
BLKS_LO = TOT_BLKS // NW
EXTRA = TOT_BLKS - NW * BLKS_LO
AGG_ROWS = 10112
ROWS_PER_TILE = AGG_ROWS // NS


def _sc_agg_body(x_hbm, idx_hbm, zeros_hbm, out_hbm,
                 src_v, dst_v, rows_v, agg_sh, gsem, isem):
    c = lax.axis_index("c")
    s = lax.axis_index("s")
    wid = c * NS + s
    r0 = s * ROWS_PER_TILE
    nblk = BLKS_LO + jnp.where(wid < EXTRA, 1, 0)
    base = (wid * BLKS_LO + jnp.minimum(wid, EXTRA)) * BLK

    pltpu.sync_copy(idx_hbm.at[0, pl.ds(base, BLK)], src_v.at[0])
    pltpu.sync_copy(idx_hbm.at[1, pl.ds(base, BLK)], dst_v.at[0])
    pltpu.async_copy(idx_hbm.at[0, pl.ds(base + BLK, BLK)], src_v.at[1],
                     isem.at[1])
    pltpu.async_copy(idx_hbm.at[1, pl.ds(base + BLK, BLK)], dst_v.at[1],
                     isem.at[1])

    pltpu.sync_copy(zeros_hbm, rows_v.at[0])
    for k in range(4):
        pltpu.sync_copy(rows_v.at[0],
                        agg_sh.at[pl.ds(r0 + k * CHUNK, CHUNK)])
    pltpu.sync_copy(rows_v.at[0, pl.ds(0, ROWS_PER_TILE - 4 * CHUNK)],
                    agg_sh.at[pl.ds(r0 + 4 * CHUNK,
                                    ROWS_PER_TILE - 4 * CHUNK)])

    pltpu.async_copy(x_hbm.at[src_v.at[0, 0]], rows_v.at[0], gsem.at[0])
    pltpu.async_copy(x_hbm.at[src_v.at[0, 1]], rows_v.at[1], gsem.at[1])
    plsc.subcore_barrier()

    def block(b, carry):
        p = b % 2
        q = 1 - p
        for r in range(BLK):
            rb = r % 2
            pltpu.make_async_copy(x_hbm.at[src_v.at[p, r]],
                                  rows_v.at[rb], gsem.at[rb]).wait()
            pltpu.sync_copy(rows_v.at[rb], agg_sh.at[dst_v.at[p, r]],
                            add=True)
            if r < BLK - 2:
                pltpu.async_copy(x_hbm.at[src_v.at[p, r + 2]],
                                 rows_v.at[rb], gsem.at[rb])
            else:
                @pl.when(b + 1 < nblk)
                def _():
                    if r == BLK - 2:
                        pltpu.make_async_copy(
                            idx_hbm.at[0, pl.ds(base + (b + 1) * BLK, BLK)],
                            src_v.at[q], isem.at[q]).wait()
                        pltpu.make_async_copy(
                            idx_hbm.at[1, pl.ds(base + (b + 1) * BLK, BLK)],
                            dst_v.at[q], isem.at[q]).wait()
                    pltpu.async_copy(x_hbm.at[src_v.at[q, r - BLK + 2]],
                                     rows_v.at[rb], gsem.at[rb])
                if r == BLK - 1:
                    @pl.when(b + 2 < nblk)
                    def _():
                        pltpu.async_copy(
                            idx_hbm.at[0, pl.ds(base + (b + 2) * BLK, BLK)],
                            src_v.at[p], isem.at[p])
                        pltpu.async_copy(
                            idx_hbm.at[1, pl.ds(base + (b + 2) * BLK, BLK)],
                            dst_v.at[p], isem.at[p])
        return carry

    lax.fori_loop(0, nblk, block, 0)
    plsc.subcore_barrier()

    pltpu.sync_copy(agg_sh.at[pl.ds(r0, ROWS_PER_TILE)],
                    out_hbm.at[c].at[pl.ds(r0, ROWS_PER_TILE)])


_sc_agg = pl.kernel(
    _sc_agg_body,
    out_type=jax.ShapeDtypeStruct((NC, AGG_ROWS, D), jnp.float32),
    mesh=plsc.VectorSubcoreMesh(core_axis_name="c", subcore_axis_name="s",
                                num_cores=NC, num_subcores=NS),
    scratch_types=[
        pltpu.VMEM((2, BLK, CHUNK), jnp.int32),
        pltpu.VMEM((2, BLK, CHUNK), jnp.int32),
        pltpu.VMEM((2, CHUNK, D), jnp.float32),
        pltpu.VMEM_SHARED((AGG_ROWS, D), jnp.float32),
        pltpu.SemaphoreType.DMA((2,)),
        pltpu.SemaphoreType.DMA((2,)),
    ],
)


def _dense1_body(aggp_ref, x_ref, wrel_ref, b_ref, wroot_ref, g_ref, be_ref,
                 out_ref):
    agg = aggp_ref[0, :N, :] + aggp_ref[1, :N, :]
    x = x_ref[...]
    y = lax.dot_general(agg, wrel_ref[...], (((1,), (1,)), ((), ())),
                        preferred_element_type=jnp.float32)
    y += lax.dot_general(x, wroot_ref[...], (((1,), (1,)), ((), ())),
                         preferred_element_type=jnp.float32)
    y += b_ref[...][None, :]
    mean = jnp.mean(y, axis=0, keepdims=True)
    var = jnp.mean((y - mean) * (y - mean), axis=0, keepdims=True)
    yn = (y - mean) * lax.rsqrt(var + 1e-5)
    yn = yn * g_ref[...][None, :] + be_ref[...][None, :]
    out_ref[...] = jnp.maximum(yn, 0.0)


_dense1 = pl.pallas_call(
    _dense1_body,
    out_shape=jax.ShapeDtypeStruct((N, HID), jnp.float32),
)


def _dense2_body(aggp_ref, h_ref, batch_ref, wrel_ref, b_ref, wroot_ref,
                 g_ref, be_ref, wc_ref, bc_ref, logits_ref, h2_ref):
    agg = aggp_ref[0, :N, :] + aggp_ref[1, :N, :]
    h = h_ref[...]
    y = lax.dot_general(agg, wrel_ref[...], (((1,), (1,)), ((), ())),
                        preferred_element_type=jnp.float32)
    y += lax.dot_general(h, wroot_ref[...], (((1,), (1,)), ((), ())),
                         preferred_element_type=jnp.float32)
    y += b_ref[...][None, :]
    mean = jnp.mean(y, axis=0, keepdims=True)
    var = jnp.mean((y - mean) * (y - mean), axis=0, keepdims=True)
    yn = (y - mean) * lax.rsqrt(var + 1e-5)
    yn = yn * g_ref[...][None, :] + be_ref[...][None, :]
    h2 = jnp.maximum(yn, 0.0)
    h2_ref[...] = h2

    gids = lax.broadcasted_iota(jnp.int32, (G, N), 0)
    onehot = jnp.where(batch_ref[...] == gids, 1.0, 0.0)
    pooled = lax.dot_general(onehot, h2, (((1,), (0,)), ((), ())),
                             preferred_element_type=jnp.float32)
    logits = lax.dot_general(pooled, wc_ref[...], (((1,), (1,)), ((), ())),
                             preferred_element_type=jnp.float32)
    logits_ref[...] = logits + bc_ref[...][None, :]


_dense2 = pl.pallas_call(
    _dense2_body,
    out_shape=[
        jax.ShapeDtypeStruct((G, OUT), jnp.float32),
        jax.ShapeDtypeStruct((N, HID), jnp.float32),
    ],
)


@jax.jit
def kernel(x, edge_index, batch, W1_rel, b1, W1_root, g1, be1,
           W2_rel, b2, W2_root, g2, be2, Wc, bc):
    idx_all = edge_index.reshape(2, TOT_CHUNKS, CHUNK)
    zeros = jnp.zeros((CHUNK, D), jnp.float32)

    aggp1 = _sc_agg(x, idx_all, zeros)
    h1 = _dense1(aggp1, x, W1_rel, b1, W1_root, g1, be1)
    aggp2 = _sc_agg(h1, idx_all, zeros)
    logits, h2 = _dense2(aggp2, h1, batch.reshape(1, N), W2_rel, b2,
                         W2_root, g2, be2, Wc, bc)
    return (logits, h2)


# speedup vs baseline: 1.1729x; 1.0058x over previous
import jax
import jax.numpy as jnp
from jax import lax
from jax.experimental import pallas as pl
from jax.experimental.pallas import tpu as pltpu
from jax.experimental.pallas import tpu_sc as plsc

N = 10000
E = 320000
D = 128
HID = 128
OUT = 10
G = 64

NC = 2
NS = 16
NW = NC * NS

CHUNK = 128
TOT_CHUNKS = E // CHUNK
BLK = 4
TOT_BLKS = TOT_CHUNKS // BLK
BLKS_LO = TOT_BLKS // NW
EXTRA = TOT_BLKS - NW * BLKS_LO
AGG_ROWS = 10112
ROWS_PER_TILE = AGG_ROWS // NS


def _sc_agg_body(x_hbm, idx_hbm, zeros_hbm, out_hbm,
                 src_v, dst_v, rows_v, agg_sh, gsem, isem):
    c = lax.axis_index("c")
    s = lax.axis_index("s")
    wid = c * NS + s
    r0 = s * ROWS_PER_TILE
    nblk = BLKS_LO + jnp.where(wid < EXTRA, 1, 0)
    base = (wid * BLKS_LO + jnp.minimum(wid, EXTRA)) * BLK

    pltpu.sync_copy(idx_hbm.at[0, pl.ds(base, BLK)], src_v.at[0])
    pltpu.sync_copy(idx_hbm.at[1, pl.ds(base, BLK)], dst_v.at[0])
    pltpu.async_copy(idx_hbm.at[0, pl.ds(base + BLK, BLK)], src_v.at[1],
                     isem.at[1])
    pltpu.async_copy(idx_hbm.at[1, pl.ds(base + BLK, BLK)], dst_v.at[1],
                     isem.at[1])

    zdesc = pltpu.make_async_copy(zeros_hbm, rows_v.at[1], gsem.at[1])
    zdesc.start()
    pltpu.async_copy(x_hbm.at[src_v.at[0, 0]], rows_v.at[0], gsem.at[0])
    zdesc.wait()
    for k in range(4):
        pltpu.sync_copy(rows_v.at[1],
                        agg_sh.at[pl.ds(r0 + k * CHUNK, CHUNK)])
    pltpu.sync_copy(rows_v.at[1, pl.ds(0, ROWS_PER_TILE - 4 * CHUNK)],
                    agg_sh.at[pl.ds(r0 + 4 * CHUNK,
                                    ROWS_PER_TILE - 4 * CHUNK)])
    pltpu.async_copy(x_hbm.at[src_v.at[0, 1]], rows_v.at[1], gsem.at[1])
    plsc.subcore_barrier()

    def block(b, carry):
        p = b % 2
        q = 1 - p
        for r in range(BLK):
            rb = r % 2
            pltpu.make_async_copy(x_hbm.at[src_v.at[p, r]],
                                  rows_v.at[rb], gsem.at[rb]).wait()
            pltpu.sync_copy(rows_v.at[rb], agg_sh.at[dst_v.at[p, r]],
                            add=True)
            if r < BLK - 2:
                pltpu.async_copy(x_hbm.at[src_v.at[p, r + 2]],
                                 rows_v.at[rb], gsem.at[rb])
            else:
                @pl.when(b + 1 < nblk)
                def _():
                    if r == BLK - 2:
                        pltpu.make_async_copy(
                            idx_hbm.at[0, pl.ds(base + (b + 1) * BLK, BLK)],
                            src_v.at[q], isem.at[q]).wait()
                        pltpu.make_async_copy(
                            idx_hbm.at[1, pl.ds(base + (b + 1) * BLK, BLK)],
                            dst_v.at[q], isem.at[q]).wait()
                    pltpu.async_copy(x_hbm.at[src_v.at[q, r - BLK + 2]],
                                     rows_v.at[rb], gsem.at[rb])
                if r == BLK - 1:
                    @pl.when(b + 2 < nblk)
                    def _():
                        pltpu.async_copy(
                            idx_hbm.at[0, pl.ds(base + (b + 2) * BLK, BLK)],
                            src_v.at[p], isem.at[p])
                        pltpu.async_copy(
                            idx_hbm.at[1, pl.ds(base + (b + 2) * BLK, BLK)],
                            dst_v.at[p], isem.at[p])
        return carry

    lax.fori_loop(0, nblk, block, 0)
    plsc.subcore_barrier()

    pltpu.sync_copy(agg_sh.at[pl.ds(r0, ROWS_PER_TILE)],
                    out_hbm.at[c].at[pl.ds(r0, ROWS_PER_TILE)])


_sc_agg = pl.kernel(
    _sc_agg_body,
    out_type=jax.ShapeDtypeStruct((NC, AGG_ROWS, D), jnp.float32),
    mesh=plsc.VectorSubcoreMesh(core_axis_name="c", subcore_axis_name="s",
                                num_cores=NC, num_subcores=NS),
    scratch_types=[
        pltpu.VMEM((2, BLK, CHUNK), jnp.int32),
        pltpu.VMEM((2, BLK, CHUNK), jnp.int32),
        pltpu.VMEM((2, CHUNK, D), jnp.float32),
        pltpu.VMEM_SHARED((AGG_ROWS, D), jnp.float32),
        pltpu.SemaphoreType.DMA((2,)),
        pltpu.SemaphoreType.DMA((2,)),
    ],
)


def _dense1_body(aggp_ref, x_ref, wrel_ref, b_ref, wroot_ref, g_ref, be_ref,
                 out_ref):
    agg = aggp_ref[0, :N, :] + aggp_ref[1, :N, :]
    x = x_ref[...]
    y = lax.dot_general(agg, wrel_ref[...], (((1,), (1,)), ((), ())),
                        preferred_element_type=jnp.float32)
    y += lax.dot_general(x, wroot_ref[...], (((1,), (1,)), ((), ())),
                         preferred_element_type=jnp.float32)
    y += b_ref[...][None, :]
    mean = jnp.mean(y, axis=0, keepdims=True)
    var = jnp.mean((y - mean) * (y - mean), axis=0, keepdims=True)
    yn = (y - mean) * lax.rsqrt(var + 1e-5)
    yn = yn * g_ref[...][None, :] + be_ref[...][None, :]
    out_ref[...] = jnp.maximum(yn, 0.0)


_dense1 = pl.pallas_call(
    _dense1_body,
    out_shape=jax.ShapeDtypeStruct((N, HID), jnp.float32),
)


def _dense2_body(aggp_ref, h_ref, batch_ref, wrel_ref, b_ref, wroot_ref,
                 g_ref, be_ref, wc_ref, bc_ref, logits_ref, h2_ref):
    agg = aggp_ref[0, :N, :] + aggp_ref[1, :N, :]
    h = h_ref[...]
    y = lax.dot_general(agg, wrel_ref[...], (((1,), (1,)), ((), ())),
                        preferred_element_type=jnp.float32)
    y += lax.dot_general(h, wroot_ref[...], (((1,), (1,)), ((), ())),
                         preferred_element_type=jnp.float32)
    y += b_ref[...][None, :]
    mean = jnp.mean(y, axis=0, keepdims=True)
    var = jnp.mean((y - mean) * (y - mean), axis=0, keepdims=True)
    yn = (y - mean) * lax.rsqrt(var + 1e-5)
    yn = yn * g_ref[...][None, :] + be_ref[...][None, :]
    h2 = jnp.maximum(yn, 0.0)
    h2_ref[...] = h2

    gids = lax.broadcasted_iota(jnp.int32, (G, N), 0)
    onehot = jnp.where(batch_ref[...] == gids, 1.0, 0.0)
    pooled = lax.dot_general(onehot, h2, (((1,), (0,)), ((), ())),
                             preferred_element_type=jnp.float32)
    logits = lax.dot_general(pooled, wc_ref[...], (((1,), (1,)), ((), ())),
                             preferred_element_type=jnp.float32)
    logits_ref[...] = logits + bc_ref[...][None, :]


_dense2 = pl.pallas_call(
    _dense2_body,
    out_shape=[
        jax.ShapeDtypeStruct((G, OUT), jnp.float32),
        jax.ShapeDtypeStruct((N, HID), jnp.float32),
    ],
)


@jax.jit
def kernel(x, edge_index, batch, W1_rel, b1, W1_root, g1, be1,
           W2_rel, b2, W2_root, g2, be2, Wc, bc):
    idx_all = edge_index.reshape(2, TOT_CHUNKS, CHUNK)
    zeros = jnp.zeros((CHUNK, D), jnp.float32)

    aggp1 = _sc_agg(x, idx_all, zeros)
    h1 = _dense1(aggp1, x, W1_rel, b1, W1_root, g1, be1)
    aggp2 = _sc_agg(h1, idx_all, zeros)
    logits, h2 = _dense2(aggp2, h1, batch.reshape(1, N), W2_rel, b2,
                         W2_root, g2, be2, Wc, bc)
    return (logits, h2)
